# baseline (speedup 1.0000x reference)
import functools

import jax
import jax.numpy as jnp
from jax import lax
from jax.experimental import pallas as pl
from jax.experimental.pallas import tpu as pltpu

E = 8
TOP_K = 2
TM = 512


def _moe_kernel(mg_ref, dom_ref, x_ref, gw_ref, *rest, num_groups):
    gu_refs = rest[:num_groups]
    dn_refs = rest[num_groups:2 * num_groups]
    out_ref = rest[2 * num_groups]

    xt = x_ref[...]

    logits = lax.dot_general(
        xt, gw_ref[...], (((1,), (1,)), ((), ())),
        preferred_element_type=jnp.float32)
    m = jnp.max(logits, axis=1, keepdims=True)
    ex = jnp.exp(logits - m)
    probs = ex / jnp.sum(ex, axis=1, keepdims=True)

    i1 = jnp.argmax(probs, axis=1)
    v1 = jnp.max(probs, axis=1)
    iota = lax.broadcasted_iota(jnp.int32, probs.shape, 1)
    masked = jnp.where(iota == i1[:, None], -jnp.inf, probs)
    i2 = jnp.argmax(masked, axis=1)
    v2 = jnp.max(masked, axis=1)
    denom = v1 + v2

    acc = None
    for g in range(num_groups):
        wg = jnp.zeros_like(v1)
        for e in range(E):
            in_g = mg_ref[e] == g
            sel = jnp.where(i1 == e, v1, 0.0) + jnp.where(i2 == e, v2, 0.0)
            wg = wg + jnp.where(in_g, sel, 0.0)
        wg = wg / denom

        gu = lax.dot_general(
            xt, gu_refs[g][0], (((1,), (1,)), ((), ())),
            preferred_element_type=jnp.float32)
        dff = gu.shape[1] // 2
        gate_h = gu[:, :dff]
        up_h = gu[:, dff:]
        h = gate_h * lax.logistic(gate_h) * up_h
        out = lax.dot_general(
            h, dn_refs[g][0], (((1,), (1,)), ((), ())),
            preferred_element_type=jnp.float32)
        term = out * wg[:, None]
        acc = term if acc is None else acc + term

    out_ref[...] = acc


def kernel(hidden_states, gate_weight, gate_up_proj, down_proj, merge_groups, dominant_experts):
    b, s, d = hidden_states.shape
    x = hidden_states.reshape(s, d)
    num_groups = dominant_experts.shape[0]
    gate_up_proj = gate_up_proj.astype(jnp.bfloat16)
    down_proj = down_proj.astype(jnp.bfloat16)
    two_dff = gate_up_proj.shape[1]
    dff = down_proj.shape[2]
    n_t = s // TM

    def gu_spec(g):
        return pl.BlockSpec((1, two_dff, d), lambda t, mg, dom: (dom[g], 0, 0))

    def dn_spec(g):
        return pl.BlockSpec((1, d, dff), lambda t, mg, dom: (dom[g], 0, 0))

    grid_spec = pltpu.PrefetchScalarGridSpec(
        num_scalar_prefetch=2,
        grid=(n_t,),
        in_specs=[
            pl.BlockSpec((TM, d), lambda t, mg, dom: (t, 0)),
            pl.BlockSpec((E, d), lambda t, mg, dom: (0, 0)),
        ] + [gu_spec(g) for g in range(num_groups)]
          + [dn_spec(g) for g in range(num_groups)],
        out_specs=pl.BlockSpec((TM, d), lambda t, mg, dom: (t, 0)),
    )

    out = pl.pallas_call(
        functools.partial(_moe_kernel, num_groups=num_groups),
        grid_spec=grid_spec,
        out_shape=jax.ShapeDtypeStruct((s, d), x.dtype),
        compiler_params=pltpu.CompilerParams(
            dimension_semantics=("arbitrary",),
        ),
    )(merge_groups, dominant_experts, x, gate_weight,
      *([gate_up_proj] * num_groups), *([down_proj] * num_groups))
    return out.reshape(b, s, d)


# speedup vs baseline: 1.5183x; 1.5183x over previous
import functools

import jax
import jax.numpy as jnp
from jax import lax
from jax.experimental import pallas as pl
from jax.experimental.pallas import tpu as pltpu

E = 8
TOP_K = 2
TM = 512


def _moe_kernel(mg_ref, dom_ref, x_ref, gw_ref, *rest, num_groups):
    gu_refs = rest[:num_groups]
    dn_refs = rest[num_groups:2 * num_groups]
    out_ref = rest[2 * num_groups]

    xt = x_ref[...]

    logits = lax.dot_general(
        xt, gw_ref[...], (((1,), (1,)), ((), ())),
        preferred_element_type=jnp.float32)
    m = jnp.max(logits, axis=1, keepdims=True)
    ex = jnp.exp(logits - m)
    probs = ex / jnp.sum(ex, axis=1, keepdims=True)

    i1 = jnp.argmax(probs, axis=1)
    v1 = jnp.max(probs, axis=1)
    iota = lax.broadcasted_iota(jnp.int32, probs.shape, 1)
    masked = jnp.where(iota == i1[:, None], -jnp.inf, probs)
    i2 = jnp.argmax(masked, axis=1)
    v2 = jnp.max(masked, axis=1)
    denom = v1 + v2

    acc = None
    for g in range(num_groups):
        wg = jnp.zeros_like(v1)
        for e in range(E):
            in_g = mg_ref[e] == g
            sel = jnp.where(i1 == e, v1, 0.0) + jnp.where(i2 == e, v2, 0.0)
            wg = wg + jnp.where(in_g, sel, 0.0)
        wg = wg / denom

        gu = lax.dot_general(
            xt, gu_refs[g][0], (((1,), (1,)), ((), ())),
            preferred_element_type=jnp.float32)
        dff = gu.shape[1] // 2
        gate_h = gu[:, :dff]
        up_h = gu[:, dff:]
        h = gate_h * lax.logistic(gate_h) * up_h
        out = lax.dot_general(
            h, dn_refs[g][0], (((1,), (1,)), ((), ())),
            preferred_element_type=jnp.float32)
        term = out * wg[:, None]
        acc = term if acc is None else acc + term

    out_ref[...] = acc


def kernel(hidden_states, gate_weight, gate_up_proj, down_proj, merge_groups, dominant_experts):
    b, s, d = hidden_states.shape
    x = hidden_states.reshape(s, d)
    num_groups = dominant_experts.shape[0]
    two_dff = gate_up_proj.shape[1]
    dff = down_proj.shape[2]
    n_t = s // TM

    def gu_spec(g):
        return pl.BlockSpec((1, two_dff, d), lambda t, mg, dom: (dom[g], 0, 0))

    def dn_spec(g):
        return pl.BlockSpec((1, d, dff), lambda t, mg, dom: (dom[g], 0, 0))

    grid_spec = pltpu.PrefetchScalarGridSpec(
        num_scalar_prefetch=2,
        grid=(n_t,),
        in_specs=[
            pl.BlockSpec((TM, d), lambda t, mg, dom: (t, 0)),
            pl.BlockSpec((E, d), lambda t, mg, dom: (0, 0)),
        ] + [gu_spec(g) for g in range(num_groups)]
          + [dn_spec(g) for g in range(num_groups)],
        out_specs=pl.BlockSpec((TM, d), lambda t, mg, dom: (t, 0)),
    )

    out = pl.pallas_call(
        functools.partial(_moe_kernel, num_groups=num_groups),
        grid_spec=grid_spec,
        out_shape=jax.ShapeDtypeStruct((s, d), x.dtype),
        compiler_params=pltpu.CompilerParams(
            dimension_semantics=("arbitrary",),
        ),
    )(merge_groups, dominant_experts, x, gate_weight,
      *([gate_up_proj] * num_groups), *([down_proj] * num_groups))
    return out.reshape(b, s, d)
